# baseline (speedup 1.0000x reference)
import functools

import jax
import jax.numpy as jnp
from jax import lax
from jax.experimental import pallas as pl
from jax.experimental.pallas import tpu as pltpu
from jax.experimental.pallas import tpu_sc as plsc

F = 128
NC = 2
NS = 16
NW = NC * NS


def _silu(x):
    return x * jax.nn.sigmoid(x)


def _make_gather(E, C):
    NCH = E // C
    NJ = -(-NCH // NW)
    NJ += NJ % 2
    mesh = plsc.VectorSubcoreMesh(core_axis_name="c", subcore_axis_name="s")

    buf_set = [
        pltpu.VMEM((C,), jnp.int32),
        pltpu.VMEM((C,), jnp.int32),
        pltpu.VMEM((C, F // 2), jnp.int32),
        pltpu.VMEM((C, 3 * F // 2), jnp.int32),
        pltpu.VMEM((C, 3 * F // 2), jnp.int32),
        pltpu.SemaphoreType.DMA,
    ]

    @functools.partial(
        pl.kernel,
        mesh=mesh,
        out_type=[
            jax.ShapeDtypeStruct((E, F // 2), jnp.int32),
            jax.ShapeDtypeStruct((E, 3 * F // 2), jnp.int32),
            jax.ShapeDtypeStruct((E, 3 * F // 2), jnp.int32),
        ],
        scratch_types=buf_set + buf_set,
        compiler_params=pltpu.CompilerParams(use_tc_tiling_on_sc=False),
    )
    def gather_kernel(s_hbm, v2_hbm, row_hbm, col_hbm,
                      srow_hbm, vrow_hbm, vcol_hbm,
                      idxr0, idxc0, sbuf0, vrbuf0, vcbuf0, sem0,
                      idxr1, idxc1, sbuf1, vrbuf1, vcbuf1, sem1):
        wid = lax.axis_index("s") * NC + lax.axis_index("c")

        def start(j, idxr, idxc, sbuf, vrbuf, vcbuf, sem):
            c = wid + NW * j

            @pl.when(c < NCH)
            def _():
                off = c * C
                pltpu.sync_copy(row_hbm.at[pl.ds(off, C)], idxr)
                pltpu.sync_copy(col_hbm.at[pl.ds(off, C)], idxc)
                pltpu.async_copy(s_hbm.at[idxr], sbuf, sem)
                pltpu.async_copy(v2_hbm.at[idxr], vrbuf, sem)
                pltpu.async_copy(v2_hbm.at[idxc], vcbuf, sem)

        def finish(j, idxr, idxc, sbuf, vrbuf, vcbuf, sem):
            c = wid + NW * j

            @pl.when(c < NCH)
            def _():
                off = c * C
                pltpu.make_async_copy(s_hbm.at[idxr], sbuf, sem).wait()
                pltpu.make_async_copy(v2_hbm.at[idxr], vrbuf, sem).wait()
                pltpu.make_async_copy(v2_hbm.at[idxc], vcbuf, sem).wait()
                pltpu.sync_copy(sbuf, srow_hbm.at[pl.ds(off, C)])
                pltpu.sync_copy(vrbuf, vrow_hbm.at[pl.ds(off, C)])
                pltpu.sync_copy(vcbuf, vcol_hbm.at[pl.ds(off, C)])

        A = (idxr0, idxc0, sbuf0, vrbuf0, vcbuf0, sem0)
        B = (idxr1, idxc1, sbuf1, vrbuf1, vcbuf1, sem1)
        start(0, *A)

        @pl.loop(0, NJ, step=2)
        def _(jj):
            start(jj + 1, *B)
            finish(jj, *A)
            start(jj + 2, *A)
            finish(jj + 1, *B)

    return gather_kernel


def _edge_body(srow_ref, rbf_ref, vrow_ref, vcol_ref,
               msW1s_ref, msW1r_ref, msb1_ref, msW2_ref, msb2_ref,
               mvW1s_ref, mvW1r_ref, mvb1_ref, mvW2_ref, mvb2_ref,
               ds_ref, dv0_ref, dv1_ref, dv2_ref):
    x = srow_ref[...]
    r = rbf_ref[...]
    h = jnp.dot(x, msW1s_ref[...], preferred_element_type=jnp.float32)
    h += jnp.dot(r, msW1r_ref[...], preferred_element_type=jnp.float32)
    h = _silu(h + msb1_ref[...]).astype(jnp.bfloat16)
    ds_ref[...] = (jnp.dot(h, msW2_ref[...], preferred_element_type=jnp.float32)
                   + msb2_ref[...])

    g = jnp.dot(x, mvW1s_ref[...], preferred_element_type=jnp.float32)
    g += jnp.dot(r, mvW1r_ref[...], preferred_element_type=jnp.float32)
    g = _silu(g + mvb1_ref[...]).astype(jnp.bfloat16)
    dvb = (jnp.dot(g, mvW2_ref[...], preferred_element_type=jnp.float32)
           + mvb2_ref[...])

    vr = vrow_ref[...].astype(jnp.float32)
    vc = vcol_ref[...].astype(jnp.float32)
    vd0 = vr[:, 0 * F:1 * F] - vc[:, 0 * F:1 * F]
    vd1 = vr[:, 1 * F:2 * F] - vc[:, 1 * F:2 * F]
    vd2 = vr[:, 2 * F:3 * F] - vc[:, 2 * F:3 * F]
    nrm = jnp.sqrt(vd0 * vd0 + vd1 * vd1 + vd2 * vd2)
    scale = dvb / (nrm + 1e-8)
    dv0_ref[...] = scale * vd0
    dv1_ref[...] = scale * vd1
    dv2_ref[...] = scale * vd2


def _make_edge_call(E, BE):
    grid = (E // BE,)

    def full(shape):
        return pl.BlockSpec(shape, lambda i: (0,) * len(shape))

    in_specs = [
        pl.BlockSpec((BE, F), lambda i: (i, 0)),
        pl.BlockSpec((BE, 16), lambda i: (i, 0)),
        pl.BlockSpec((BE, 3 * F), lambda i: (i, 0)),
        pl.BlockSpec((BE, 3 * F), lambda i: (i, 0)),
        full((F, F)), full((16, F)), full((1, F)), full((F, F)), full((1, F)),
        full((F, F)), full((16, F)), full((1, F)), full((F, F)), full((1, F)),
    ]
    out_specs = [pl.BlockSpec((BE, F), lambda i: (i, 0))] * 4
    out_shape = [jax.ShapeDtypeStruct((E, F), jnp.float32)] * 4
    return pl.pallas_call(
        _edge_body, grid=grid, in_specs=in_specs, out_specs=out_specs,
        out_shape=out_shape)


def _make_scatter(E, N, C):
    NCH = E // C
    NJ = -(-NCH // NW)
    NJ += NJ % 2
    RSUB = (N // NS) // 8 * 8
    RREM = N - NS * RSUB
    mesh = plsc.VectorSubcoreMesh(core_axis_name="c", subcore_axis_name="s")

    buf_set = [
        pltpu.VMEM((C,), jnp.int32),
        pltpu.VMEM((C, F), jnp.float32),
        pltpu.SemaphoreType.DMA,
    ]

    @functools.partial(
        pl.kernel,
        mesh=mesh,
        out_type=[jax.ShapeDtypeStruct((NC, N, F), jnp.float32)] * 4,
        scratch_types=buf_set + buf_set + [
            pltpu.VMEM_SHARED((N, F), jnp.float32),
        ],
    )
    def scatter_kernel(ds_hbm, dv0_hbm, dv1_hbm, dv2_hbm, col_hbm, zero_hbm,
                       o0, o1, o2, o3,
                       idx0, dbuf0, sem0, idx1, dbuf1, sem1, acc):
        cid = lax.axis_index("c")
        sid = lax.axis_index("s")
        wid = sid * NC + cid
        rbase = sid * RSUB

        for data_hbm, out_hbm in ((ds_hbm, o0), (dv0_hbm, o1),
                                  (dv1_hbm, o2), (dv2_hbm, o3)):
            pltpu.sync_copy(zero_hbm.at[pl.ds(0, RSUB)],
                            acc.at[pl.ds(rbase, RSUB)])

            @pl.when(sid == NS - 1)
            def _():
                pltpu.sync_copy(zero_hbm.at[pl.ds(0, RREM)],
                                acc.at[pl.ds(NS * RSUB, RREM)])

            plsc.subcore_barrier()

            def start(j, idx, dbuf, sem):
                c = wid + NW * j

                @pl.when(c < NCH)
                def _():
                    off = c * C
                    pltpu.async_copy(col_hbm.at[pl.ds(off, C)], idx, sem)
                    pltpu.async_copy(data_hbm.at[pl.ds(off, C)], dbuf, sem)

            def finish(j, idx, dbuf, sem):
                c = wid + NW * j

                @pl.when(c < NCH)
                def _():
                    pltpu.make_async_copy(col_hbm.at[pl.ds(0, C)], idx,
                                          sem).wait()
                    pltpu.make_async_copy(data_hbm.at[pl.ds(0, C)], dbuf,
                                          sem).wait()
                    pltpu.sync_copy(dbuf, acc.at[idx], add=True)

            A = (idx0, dbuf0, sem0)
            B = (idx1, dbuf1, sem1)
            start(0, *A)

            @pl.loop(0, NJ, step=2)
            def _(jj):
                start(jj + 1, *B)
                finish(jj, *A)
                start(jj + 2, *A)
                finish(jj + 1, *B)

            plsc.subcore_barrier()

            pltpu.sync_copy(acc.at[pl.ds(rbase, RSUB)],
                            out_hbm.at[cid].at[pl.ds(rbase, RSUB)])

            @pl.when(sid == NS - 1)
            def _():
                pltpu.sync_copy(acc.at[pl.ds(NS * RSUB, RREM)],
                                out_hbm.at[cid].at[pl.ds(NS * RSUB, RREM)])

            plsc.subcore_barrier()

    return scatter_kernel


def _node_body(s_ref, v2_ref, a0_ref, a1_ref, a2_ref, a3_ref,
               usW1a_ref, usW1b_ref, usb1_ref, usW2_ref, usb2_ref,
               uvW1a_ref, uvW1b_ref, uvb1_ref, uvW2_ref, uvb2_ref,
               sout_ref, vout_ref):
    sv = s_ref[...]
    dsa = a0_ref[0] + a0_ref[1]
    h = jnp.dot(sv, usW1a_ref[...], preferred_element_type=jnp.float32)
    h += jnp.dot(dsa, usW1b_ref[...], preferred_element_type=jnp.float32)
    h = _silu(h + usb1_ref[...])
    sout_ref[...] = sv + (jnp.dot(h, usW2_ref[...],
                                  preferred_element_type=jnp.float32)
                          + usb2_ref[...])

    for k, ak_ref in enumerate((a1_ref, a2_ref, a3_ref)):
        vk = v2_ref[:, k * F:(k + 1) * F]
        dvk = ak_ref[0] + ak_ref[1]
        g = jnp.dot(vk, uvW1a_ref[...], preferred_element_type=jnp.float32)
        g += jnp.dot(dvk, uvW1b_ref[...], preferred_element_type=jnp.float32)
        g = _silu(g + uvb1_ref[...])
        vout_ref[:, k * F:(k + 1) * F] = vk + (
            jnp.dot(g, uvW2_ref[...], preferred_element_type=jnp.float32)
            + uvb2_ref[...])


def _make_node_call(N, BN):
    grid = (N // BN,)

    def full(shape):
        return pl.BlockSpec(shape, lambda i: (0,) * len(shape))

    in_specs = [
        pl.BlockSpec((BN, F), lambda i: (i, 0)),
        pl.BlockSpec((BN, 3 * F), lambda i: (i, 0)),
        pl.BlockSpec((NC, BN, F), lambda i: (0, i, 0)),
        pl.BlockSpec((NC, BN, F), lambda i: (0, i, 0)),
        pl.BlockSpec((NC, BN, F), lambda i: (0, i, 0)),
        pl.BlockSpec((NC, BN, F), lambda i: (0, i, 0)),
        full((F, F)), full((F, F)), full((1, F)), full((F, F)), full((1, F)),
        full((F, F)), full((F, F)), full((1, F)), full((F, F)), full((1, F)),
    ]
    out_specs = [
        pl.BlockSpec((BN, F), lambda i: (i, 0)),
        pl.BlockSpec((BN, 3 * F), lambda i: (i, 0)),
    ]
    out_shape = [
        jax.ShapeDtypeStruct((N, F), jnp.float32),
        jax.ShapeDtypeStruct((N, 3 * F), jnp.float32),
    ]
    return pl.pallas_call(
        _node_body, grid=grid, in_specs=in_specs, out_specs=out_specs,
        out_shape=out_shape)


def kernel(s, v, edge_index, rbf,
           msW1, msb1, msW2, msb2,
           mvW1, mvb1, mvW2, mvb2,
           usW1, usb1, usW2, usb2,
           uvW1, uvb1, uvW2, uvb2):
    N = s.shape[0]
    E = edge_index.shape[1]
    v2 = v.reshape(N, 3 * F)
    row = edge_index[0]
    col = edge_index[1]

    bf = jnp.bfloat16

    def pack(x):
        xb = x.astype(bf)
        return jax.lax.bitcast_convert_type(
            xb.reshape(x.shape[0], -1, 2), jnp.int32)

    def unpack(xi):
        return jax.lax.bitcast_convert_type(xi, bf).reshape(xi.shape[0], -1)

    srow_p, vrow_p, vcol_p = _make_gather(E, 64)(pack(s), pack(v2), row, col)
    srow, vrow, vcol = unpack(srow_p), unpack(vrow_p), unpack(vcol_p)

    def t(W):
        return W.T

    ds, dv0, dv1, dv2 = _make_edge_call(E, 1600)(
        srow, rbf.astype(bf), vrow, vcol,
        t(msW1)[:F].astype(bf), t(msW1)[F:].astype(bf), msb1.reshape(1, F),
        t(msW2).astype(bf), msb2.reshape(1, F),
        t(mvW1)[:F].astype(bf), t(mvW1)[F:].astype(bf), mvb1.reshape(1, F),
        t(mvW2).astype(bf), mvb2.reshape(1, F),
    )

    zero = jnp.zeros(((N // NS) // 8 * 8, F), jnp.float32)
    a0, a1, a2, a3 = _make_scatter(E, N, 128)(ds, dv0, dv1, dv2, col, zero)

    s_out, v2_out = _make_node_call(N, 2000)(
        s, v2, a0, a1, a2, a3,
        t(usW1)[:F], t(usW1)[F:], usb1.reshape(1, F), t(usW2), usb2.reshape(1, F),
        t(uvW1)[:F], t(uvW1)[F:], uvb1.reshape(1, F), t(uvW2), uvb2.reshape(1, F),
    )
    return s_out, v2_out.reshape(N, 3, F)


# speedup vs baseline: 4.1390x; 4.1390x over previous
import functools

import jax
import jax.numpy as jnp
from jax import lax
from jax.experimental import pallas as pl
from jax.experimental.pallas import tpu as pltpu
from jax.experimental.pallas import tpu_sc as plsc

F = 128
NC = 2
NS = 16
NW = NC * NS


def _silu(x):
    return x * jax.nn.sigmoid(x)


def _make_gather(E, C):
    NCH = E // C
    NJ = -(-NCH // NW)
    NJ += NJ % 2
    mesh = plsc.VectorSubcoreMesh(core_axis_name="c", subcore_axis_name="s")

    W = 2 * F
    buf_set = [
        pltpu.VMEM((C,), jnp.int32),
        pltpu.VMEM((C,), jnp.int32),
        pltpu.VMEM((C, W), jnp.int32),
        pltpu.VMEM((C, W), jnp.int32),
        pltpu.SemaphoreType.DMA,
    ]

    @functools.partial(
        pl.kernel,
        mesh=mesh,
        out_type=[
            jax.ShapeDtypeStruct((E, W), jnp.int32),
            jax.ShapeDtypeStruct((E, W), jnp.int32),
        ],
        scratch_types=buf_set + buf_set,
    )
    def gather_kernel(tbl_hbm, row_hbm, col_hbm,
                      rowd_hbm, cold_hbm,
                      idxr0, idxc0, rbuf0, cbuf0, sem0,
                      idxr1, idxc1, rbuf1, cbuf1, sem1):
        wid = lax.axis_index("s") * NC + lax.axis_index("c")

        def start(j, idxr, idxc, rbuf, cbuf, sem):
            c = wid + NW * j

            @pl.when(c < NCH)
            def _():
                off = c * C
                pltpu.sync_copy(row_hbm.at[pl.ds(off, C)], idxr)
                pltpu.sync_copy(col_hbm.at[pl.ds(off, C)], idxc)
                pltpu.async_copy(tbl_hbm.at[idxr], rbuf, sem)
                pltpu.async_copy(tbl_hbm.at[idxc], cbuf, sem)

        def finish(j, idxr, idxc, rbuf, cbuf, sem):
            c = wid + NW * j

            @pl.when(c < NCH)
            def _():
                off = c * C
                pltpu.make_async_copy(tbl_hbm.at[idxr], rbuf, sem).wait()
                pltpu.make_async_copy(tbl_hbm.at[idxc], cbuf, sem).wait()
                pltpu.sync_copy(rbuf, rowd_hbm.at[pl.ds(off, C)])
                pltpu.sync_copy(cbuf, cold_hbm.at[pl.ds(off, C)])

        A = (idxr0, idxc0, rbuf0, cbuf0, sem0)
        B = (idxr1, idxc1, rbuf1, cbuf1, sem1)
        start(0, *A)

        @pl.loop(0, NJ, step=2)
        def _(jj):
            start(jj + 1, *B)
            finish(jj, *A)
            start(jj + 2, *A)
            finish(jj + 1, *B)

    return gather_kernel


def _lo(x):
    return jax.lax.bitcast_convert_type(x << 16, jnp.float32)


def _hi(x):
    return jax.lax.bitcast_convert_type(x & jnp.int32(-65536), jnp.float32)


def _edge_body(rowd_ref, cold_ref, rbf_ref,
               msW1s_ref, msW1r_ref, msb1_ref, msW2_ref, msb2_ref,
               mvW1s_ref, mvW1r_ref, mvb1_ref, mvW2_ref, mvb2_ref,
               ds_ref, dv0_ref, dv1_ref, dv2_ref):
    rowa = rowd_ref[:, :F]
    rowb = rowd_ref[:, F:]
    cola = cold_ref[:, :F]
    colb = cold_ref[:, F:]

    x = _lo(rowa).astype(jnp.bfloat16)
    r = rbf_ref[...]
    h = jnp.dot(x, msW1s_ref[...], preferred_element_type=jnp.float32)
    h += jnp.dot(r, msW1r_ref[...], preferred_element_type=jnp.float32)
    h = _silu(h + msb1_ref[...]).astype(jnp.bfloat16)
    ds_ref[...] = (jnp.dot(h, msW2_ref[...], preferred_element_type=jnp.float32)
                   + msb2_ref[...])

    g = jnp.dot(x, mvW1s_ref[...], preferred_element_type=jnp.float32)
    g += jnp.dot(r, mvW1r_ref[...], preferred_element_type=jnp.float32)
    g = _silu(g + mvb1_ref[...]).astype(jnp.bfloat16)
    dvb = (jnp.dot(g, mvW2_ref[...], preferred_element_type=jnp.float32)
           + mvb2_ref[...])

    vd0 = _hi(rowa) - _hi(cola)
    vd1 = _lo(rowb) - _lo(colb)
    vd2 = _hi(rowb) - _hi(colb)
    nrm = jnp.sqrt(vd0 * vd0 + vd1 * vd1 + vd2 * vd2)
    scale = dvb / (nrm + 1e-8)
    dv0_ref[...] = scale * vd0
    dv1_ref[...] = scale * vd1
    dv2_ref[...] = scale * vd2


def _make_edge_call(E, BE):
    grid = (E // BE,)

    def full(shape):
        return pl.BlockSpec(shape, lambda i: (0,) * len(shape))

    in_specs = [
        pl.BlockSpec((BE, 2 * F), lambda i: (i, 0)),
        pl.BlockSpec((BE, 2 * F), lambda i: (i, 0)),
        pl.BlockSpec((BE, 16), lambda i: (i, 0)),
        full((F, F)), full((16, F)), full((1, F)), full((F, F)), full((1, F)),
        full((F, F)), full((16, F)), full((1, F)), full((F, F)), full((1, F)),
    ]
    out_specs = [pl.BlockSpec((BE, F), lambda i: (i, 0))] * 4
    out_shape = [jax.ShapeDtypeStruct((E, F), jnp.float32)] * 4
    return pl.pallas_call(
        _edge_body, grid=grid, in_specs=in_specs, out_specs=out_specs,
        out_shape=out_shape)


def _make_scatter(E, N, C):
    NCH = E // C
    NJ = -(-NCH // NW)
    NJ += NJ % 2
    RSUB = (N // NS) // 8 * 8
    RREM = N - NS * RSUB
    mesh = plsc.VectorSubcoreMesh(core_axis_name="c", subcore_axis_name="s")

    buf_set = [
        pltpu.VMEM((C,), jnp.int32),
        pltpu.VMEM((C, F), jnp.float32),
        pltpu.SemaphoreType.DMA,
    ]

    @functools.partial(
        pl.kernel,
        mesh=mesh,
        out_type=[jax.ShapeDtypeStruct((NC, N, F), jnp.float32)] * 4,
        scratch_types=buf_set + buf_set + [
            pltpu.VMEM_SHARED((N, F), jnp.float32),
        ],
    )
    def scatter_kernel(ds_hbm, dv0_hbm, dv1_hbm, dv2_hbm, col_hbm, zero_hbm,
                       o0, o1, o2, o3,
                       idx0, dbuf0, sem0, idx1, dbuf1, sem1, acc):
        cid = lax.axis_index("c")
        sid = lax.axis_index("s")
        wid = sid * NC + cid
        rbase = sid * RSUB

        for data_hbm, out_hbm in ((ds_hbm, o0), (dv0_hbm, o1),
                                  (dv1_hbm, o2), (dv2_hbm, o3)):
            pltpu.sync_copy(zero_hbm.at[pl.ds(0, RSUB)],
                            acc.at[pl.ds(rbase, RSUB)])

            @pl.when(sid == NS - 1)
            def _():
                pltpu.sync_copy(zero_hbm.at[pl.ds(0, RREM)],
                                acc.at[pl.ds(NS * RSUB, RREM)])

            plsc.subcore_barrier()

            def start(j, idx, dbuf, sem):
                c = wid + NW * j

                @pl.when(c < NCH)
                def _():
                    off = c * C
                    pltpu.async_copy(col_hbm.at[pl.ds(off, C)], idx, sem)
                    pltpu.async_copy(data_hbm.at[pl.ds(off, C)], dbuf, sem)

            def finish(j, idx, dbuf, sem):
                c = wid + NW * j

                @pl.when(c < NCH)
                def _():
                    pltpu.make_async_copy(col_hbm.at[pl.ds(0, C)], idx,
                                          sem).wait()
                    pltpu.make_async_copy(data_hbm.at[pl.ds(0, C)], dbuf,
                                          sem).wait()
                    pltpu.sync_copy(dbuf, acc.at[idx], add=True)

            A = (idx0, dbuf0, sem0)
            B = (idx1, dbuf1, sem1)
            start(0, *A)

            @pl.loop(0, NJ, step=2)
            def _(jj):
                start(jj + 1, *B)
                finish(jj, *A)
                start(jj + 2, *A)
                finish(jj + 1, *B)

            plsc.subcore_barrier()

            pltpu.sync_copy(acc.at[pl.ds(rbase, RSUB)],
                            out_hbm.at[cid].at[pl.ds(rbase, RSUB)])

            @pl.when(sid == NS - 1)
            def _():
                pltpu.sync_copy(acc.at[pl.ds(NS * RSUB, RREM)],
                                out_hbm.at[cid].at[pl.ds(NS * RSUB, RREM)])

            plsc.subcore_barrier()

    return scatter_kernel


def _node_body(s_ref, v2_ref, a0_ref, a1_ref, a2_ref, a3_ref,
               usW1a_ref, usW1b_ref, usb1_ref, usW2_ref, usb2_ref,
               uvW1a_ref, uvW1b_ref, uvb1_ref, uvW2_ref, uvb2_ref,
               sout_ref, vout_ref):
    sv = s_ref[...]
    dsa = a0_ref[0] + a0_ref[1]
    h = jnp.dot(sv, usW1a_ref[...], preferred_element_type=jnp.float32)
    h += jnp.dot(dsa, usW1b_ref[...], preferred_element_type=jnp.float32)
    h = _silu(h + usb1_ref[...])
    sout_ref[...] = sv + (jnp.dot(h, usW2_ref[...],
                                  preferred_element_type=jnp.float32)
                          + usb2_ref[...])

    for k, ak_ref in enumerate((a1_ref, a2_ref, a3_ref)):
        vk = v2_ref[:, k * F:(k + 1) * F]
        dvk = ak_ref[0] + ak_ref[1]
        g = jnp.dot(vk, uvW1a_ref[...], preferred_element_type=jnp.float32)
        g += jnp.dot(dvk, uvW1b_ref[...], preferred_element_type=jnp.float32)
        g = _silu(g + uvb1_ref[...])
        vout_ref[:, k * F:(k + 1) * F] = vk + (
            jnp.dot(g, uvW2_ref[...], preferred_element_type=jnp.float32)
            + uvb2_ref[...])


def _make_node_call(N, BN):
    grid = (N // BN,)

    def full(shape):
        return pl.BlockSpec(shape, lambda i: (0,) * len(shape))

    in_specs = [
        pl.BlockSpec((BN, F), lambda i: (i, 0)),
        pl.BlockSpec((BN, 3 * F), lambda i: (i, 0)),
        pl.BlockSpec((NC, BN, F), lambda i: (0, i, 0)),
        pl.BlockSpec((NC, BN, F), lambda i: (0, i, 0)),
        pl.BlockSpec((NC, BN, F), lambda i: (0, i, 0)),
        pl.BlockSpec((NC, BN, F), lambda i: (0, i, 0)),
        full((F, F)), full((F, F)), full((1, F)), full((F, F)), full((1, F)),
        full((F, F)), full((F, F)), full((1, F)), full((F, F)), full((1, F)),
    ]
    out_specs = [
        pl.BlockSpec((BN, F), lambda i: (i, 0)),
        pl.BlockSpec((BN, 3 * F), lambda i: (i, 0)),
    ]
    out_shape = [
        jax.ShapeDtypeStruct((N, F), jnp.float32),
        jax.ShapeDtypeStruct((N, 3 * F), jnp.float32),
    ]
    return pl.pallas_call(
        _node_body, grid=grid, in_specs=in_specs, out_specs=out_specs,
        out_shape=out_shape)


def kernel(s, v, edge_index, rbf,
           msW1, msb1, msW2, msb2,
           mvW1, mvb1, mvW2, mvb2,
           usW1, usb1, usW2, usb2,
           uvW1, uvb1, uvW2, uvb2):
    N = s.shape[0]
    E = edge_index.shape[1]
    v2 = v.reshape(N, 3 * F)
    row = edge_index[0]
    col = edge_index[1]

    bf = jnp.bfloat16
    sb = s.astype(bf)
    vb = v.astype(bf)
    tbl = jnp.concatenate([
        jax.lax.bitcast_convert_type(
            jnp.stack([sb, vb[:, 0]], axis=-1), jnp.int32),
        jax.lax.bitcast_convert_type(
            jnp.stack([vb[:, 1], vb[:, 2]], axis=-1), jnp.int32),
    ], axis=1)

    rowdat, coldat = _make_gather(E, 80)(tbl, row, col)

    def t(W):
        return W.T

    ds, dv0, dv1, dv2 = _make_edge_call(E, 1600)(
        rowdat, coldat, rbf.astype(bf),
        t(msW1)[:F].astype(bf), t(msW1)[F:].astype(bf), msb1.reshape(1, F),
        t(msW2).astype(bf), msb2.reshape(1, F),
        t(mvW1)[:F].astype(bf), t(mvW1)[F:].astype(bf), mvb1.reshape(1, F),
        t(mvW2).astype(bf), mvb2.reshape(1, F),
    )

    zero = jnp.zeros(((N // NS) // 8 * 8, F), jnp.float32)
    a0, a1, a2, a3 = _make_scatter(E, N, 128)(ds, dv0, dv1, dv2, col, zero)

    s_out, v2_out = _make_node_call(N, 2000)(
        s, v2, a0, a1, a2, a3,
        t(usW1)[:F], t(usW1)[F:], usb1.reshape(1, F), t(usW2), usb2.reshape(1, F),
        t(uvW1)[:F], t(uvW1)[F:], uvb1.reshape(1, F), t(uvW2), uvb2.reshape(1, F),
    )
    return s_out, v2_out.reshape(N, 3, F)


# speedup vs baseline: 4.1657x; 1.0064x over previous
import functools

import jax
import jax.numpy as jnp
from jax import lax
from jax.experimental import pallas as pl
from jax.experimental.pallas import tpu as pltpu
from jax.experimental.pallas import tpu_sc as plsc

F = 128
NC = 2
NS = 16
NW = NC * NS


def _silu(x):
    return x * jax.nn.sigmoid(x)


def _make_gather(E, C):
    NCH = E // C
    NJ = -(-NCH // NW)
    NJ += NJ % 2
    mesh = plsc.VectorSubcoreMesh(core_axis_name="c", subcore_axis_name="s")

    W = 2 * F
    buf_set = [
        pltpu.VMEM((C,), jnp.int32),
        pltpu.VMEM((C,), jnp.int32),
        pltpu.VMEM((C, W), jnp.int32),
        pltpu.VMEM((C, W), jnp.int32),
        pltpu.SemaphoreType.DMA,
    ]

    @functools.partial(
        pl.kernel,
        mesh=mesh,
        out_type=[
            jax.ShapeDtypeStruct((E, W), jnp.int32),
            jax.ShapeDtypeStruct((E, W), jnp.int32),
        ],
        scratch_types=buf_set + buf_set,
    )
    def gather_kernel(tbl_hbm, row_hbm, col_hbm,
                      rowd_hbm, cold_hbm,
                      idxr0, idxc0, rbuf0, cbuf0, sem0,
                      idxr1, idxc1, rbuf1, cbuf1, sem1):
        wid = lax.axis_index("s") * NC + lax.axis_index("c")

        def start(j, idxr, idxc, rbuf, cbuf, sem):
            c = wid + NW * j

            @pl.when(c < NCH)
            def _():
                off = c * C
                pltpu.sync_copy(row_hbm.at[pl.ds(off, C)], idxr)
                pltpu.sync_copy(col_hbm.at[pl.ds(off, C)], idxc)
                pltpu.async_copy(tbl_hbm.at[idxr], rbuf, sem)
                pltpu.async_copy(tbl_hbm.at[idxc], cbuf, sem)

        def finish(j, idxr, idxc, rbuf, cbuf, sem):
            c = wid + NW * j

            @pl.when(c < NCH)
            def _():
                off = c * C
                pltpu.make_async_copy(tbl_hbm.at[idxr], rbuf, sem).wait()
                pltpu.make_async_copy(tbl_hbm.at[idxc], cbuf, sem).wait()
                pltpu.sync_copy(rbuf, rowd_hbm.at[pl.ds(off, C)])
                pltpu.sync_copy(cbuf, cold_hbm.at[pl.ds(off, C)])

        A = (idxr0, idxc0, rbuf0, cbuf0, sem0)
        B = (idxr1, idxc1, rbuf1, cbuf1, sem1)
        start(0, *A)

        @pl.loop(0, NJ, step=2)
        def _(jj):
            start(jj + 1, *B)
            finish(jj, *A)
            start(jj + 2, *A)
            finish(jj + 1, *B)

    return gather_kernel


def _lo(x):
    return jax.lax.bitcast_convert_type(x << 16, jnp.float32)


def _hi(x):
    return jax.lax.bitcast_convert_type(x & jnp.int32(-65536), jnp.float32)


def _edge_body(rowd_ref, cold_ref, rbf_ref,
               msW1s_ref, msW1r_ref, msb1_ref, msW2_ref, msb2_ref,
               mvW1s_ref, mvW1r_ref, mvb1_ref, mvW2_ref, mvb2_ref,
               ds_ref, dv0_ref, dv1_ref, dv2_ref):
    rowa = rowd_ref[:, :F]
    rowb = rowd_ref[:, F:]
    cola = cold_ref[:, :F]
    colb = cold_ref[:, F:]

    x = _lo(rowa).astype(jnp.bfloat16)
    r = rbf_ref[...]
    h = jnp.dot(x, msW1s_ref[...], preferred_element_type=jnp.float32)
    h += jnp.dot(r, msW1r_ref[...], preferred_element_type=jnp.float32)
    h = _silu(h + msb1_ref[...]).astype(jnp.bfloat16)
    ds_ref[...] = (jnp.dot(h, msW2_ref[...], preferred_element_type=jnp.float32)
                   + msb2_ref[...])

    g = jnp.dot(x, mvW1s_ref[...], preferred_element_type=jnp.float32)
    g += jnp.dot(r, mvW1r_ref[...], preferred_element_type=jnp.float32)
    g = _silu(g + mvb1_ref[...]).astype(jnp.bfloat16)
    dvb = (jnp.dot(g, mvW2_ref[...], preferred_element_type=jnp.float32)
           + mvb2_ref[...])

    vd0 = _hi(rowa) - _hi(cola)
    vd1 = _lo(rowb) - _lo(colb)
    vd2 = _hi(rowb) - _hi(colb)
    nrm = jnp.sqrt(vd0 * vd0 + vd1 * vd1 + vd2 * vd2)
    scale = dvb / (nrm + 1e-8)
    dv0_ref[...] = scale * vd0
    dv1_ref[...] = scale * vd1
    dv2_ref[...] = scale * vd2


def _make_edge_call(E, BE):
    grid = (E // BE,)

    def full(shape):
        return pl.BlockSpec(shape, lambda i: (0,) * len(shape))

    in_specs = [
        pl.BlockSpec((BE, 2 * F), lambda i: (i, 0)),
        pl.BlockSpec((BE, 2 * F), lambda i: (i, 0)),
        pl.BlockSpec((BE, 16), lambda i: (i, 0)),
        full((F, F)), full((16, F)), full((1, F)), full((F, F)), full((1, F)),
        full((F, F)), full((16, F)), full((1, F)), full((F, F)), full((1, F)),
    ]
    out_specs = [pl.BlockSpec((BE, F), lambda i: (i, 0))] * 4
    out_shape = [jax.ShapeDtypeStruct((E, F), jnp.float32)] * 4
    return pl.pallas_call(
        _edge_body, grid=grid, in_specs=in_specs, out_specs=out_specs,
        out_shape=out_shape)


def _make_scatter(EH, N, C):
    NCH = EH // C
    NJ = -(-NCH // NW)
    NJ += NJ % 2
    RSUB = (N // NS) // 8 * 8
    RREM = N - NS * RSUB
    mesh = plsc.VectorSubcoreMesh(core_axis_name="c", subcore_axis_name="s")

    buf_set = [
        pltpu.VMEM((C,), jnp.int32),
        pltpu.VMEM((C, F), jnp.float32),
        pltpu.SemaphoreType.DMA,
    ]

    @functools.partial(
        pl.kernel,
        mesh=mesh,
        out_type=[jax.ShapeDtypeStruct((NC, N, F), jnp.float32)] * 4,
        scratch_types=buf_set + buf_set + [
            pltpu.VMEM_SHARED((N, F), jnp.float32),
        ],
    )
    def scatter_kernel(d0a, d0b, d1a, d1b, d2a, d2b, d3a, d3b,
                       cola_hbm, colb_hbm, zero_hbm,
                       o0, o1, o2, o3,
                       idx0, dbuf0, sem0, idx1, dbuf1, sem1, acc):
        cid = lax.axis_index("c")
        sid = lax.axis_index("s")
        wid = sid * NC + cid
        rbase = sid * RSUB

        def accumulate(data_hbm, col_hbm):
            def start(j, idx, dbuf, sem):
                c = wid + NW * j

                @pl.when(c < NCH)
                def _():
                    off = c * C
                    pltpu.async_copy(col_hbm.at[pl.ds(off, C)], idx, sem)
                    pltpu.async_copy(data_hbm.at[pl.ds(off, C)], dbuf, sem)

            def finish(j, idx, dbuf, sem):
                c = wid + NW * j

                @pl.when(c < NCH)
                def _():
                    pltpu.make_async_copy(col_hbm.at[pl.ds(0, C)], idx,
                                          sem).wait()
                    pltpu.make_async_copy(data_hbm.at[pl.ds(0, C)], dbuf,
                                          sem).wait()
                    pltpu.sync_copy(dbuf, acc.at[idx], add=True)

            A = (idx0, dbuf0, sem0)
            B = (idx1, dbuf1, sem1)
            start(0, *A)

            @pl.loop(0, NJ, step=2)
            def _(jj):
                start(jj + 1, *B)
                finish(jj, *A)
                start(jj + 2, *A)
                finish(jj + 1, *B)

        for (da, db), out_hbm in (((d0a, d0b), o0), ((d1a, d1b), o1),
                                  ((d2a, d2b), o2), ((d3a, d3b), o3)):
            pltpu.sync_copy(zero_hbm.at[pl.ds(0, RSUB)],
                            acc.at[pl.ds(rbase, RSUB)])

            @pl.when(sid == NS - 1)
            def _():
                pltpu.sync_copy(zero_hbm.at[pl.ds(0, RREM)],
                                acc.at[pl.ds(NS * RSUB, RREM)])

            plsc.subcore_barrier()
            accumulate(da, cola_hbm)
            accumulate(db, colb_hbm)
            plsc.subcore_barrier()

            pltpu.sync_copy(acc.at[pl.ds(rbase, RSUB)],
                            out_hbm.at[cid].at[pl.ds(rbase, RSUB)])

            @pl.when(sid == NS - 1)
            def _():
                pltpu.sync_copy(acc.at[pl.ds(NS * RSUB, RREM)],
                                out_hbm.at[cid].at[pl.ds(NS * RSUB, RREM)])

            plsc.subcore_barrier()

    return scatter_kernel


def _node_body(s_ref, v2_ref, a0_ref, a1_ref, a2_ref, a3_ref,
               usW1a_ref, usW1b_ref, usb1_ref, usW2_ref, usb2_ref,
               uvW1a_ref, uvW1b_ref, uvb1_ref, uvW2_ref, uvb2_ref,
               sout_ref, vout_ref):
    sv = s_ref[...]
    dsa = a0_ref[0] + a0_ref[1]
    h = jnp.dot(sv, usW1a_ref[...], preferred_element_type=jnp.float32)
    h += jnp.dot(dsa, usW1b_ref[...], preferred_element_type=jnp.float32)
    h = _silu(h + usb1_ref[...])
    sout_ref[...] = sv + (jnp.dot(h, usW2_ref[...],
                                  preferred_element_type=jnp.float32)
                          + usb2_ref[...])

    for k, ak_ref in enumerate((a1_ref, a2_ref, a3_ref)):
        vk = v2_ref[:, k * F:(k + 1) * F]
        dvk = ak_ref[0] + ak_ref[1]
        g = jnp.dot(vk, uvW1a_ref[...], preferred_element_type=jnp.float32)
        g += jnp.dot(dvk, uvW1b_ref[...], preferred_element_type=jnp.float32)
        g = _silu(g + uvb1_ref[...])
        vout_ref[:, k * F:(k + 1) * F] = vk + (
            jnp.dot(g, uvW2_ref[...], preferred_element_type=jnp.float32)
            + uvb2_ref[...])


def _make_node_call(N, BN):
    grid = (N // BN,)

    def full(shape):
        return pl.BlockSpec(shape, lambda i: (0,) * len(shape))

    in_specs = [
        pl.BlockSpec((BN, F), lambda i: (i, 0)),
        pl.BlockSpec((BN, 3 * F), lambda i: (i, 0)),
        pl.BlockSpec((NC, BN, F), lambda i: (0, i, 0)),
        pl.BlockSpec((NC, BN, F), lambda i: (0, i, 0)),
        pl.BlockSpec((NC, BN, F), lambda i: (0, i, 0)),
        pl.BlockSpec((NC, BN, F), lambda i: (0, i, 0)),
        full((F, F)), full((F, F)), full((1, F)), full((F, F)), full((1, F)),
        full((F, F)), full((F, F)), full((1, F)), full((F, F)), full((1, F)),
    ]
    out_specs = [
        pl.BlockSpec((BN, F), lambda i: (i, 0)),
        pl.BlockSpec((BN, 3 * F), lambda i: (i, 0)),
    ]
    out_shape = [
        jax.ShapeDtypeStruct((N, F), jnp.float32),
        jax.ShapeDtypeStruct((N, 3 * F), jnp.float32),
    ]
    return pl.pallas_call(
        _node_body, grid=grid, in_specs=in_specs, out_specs=out_specs,
        out_shape=out_shape)


def kernel(s, v, edge_index, rbf,
           msW1, msb1, msW2, msb2,
           mvW1, mvb1, mvW2, mvb2,
           usW1, usb1, usW2, usb2,
           uvW1, uvb1, uvW2, uvb2):
    N = s.shape[0]
    E = edge_index.shape[1]
    v2 = v.reshape(N, 3 * F)
    row = edge_index[0]
    col = edge_index[1]

    bf = jnp.bfloat16
    sb = s.astype(bf)
    vb = v.astype(bf)
    tbl = jnp.concatenate([
        jax.lax.bitcast_convert_type(
            jnp.stack([sb, vb[:, 0]], axis=-1), jnp.int32),
        jax.lax.bitcast_convert_type(
            jnp.stack([vb[:, 1], vb[:, 2]], axis=-1), jnp.int32),
    ], axis=1)

    def t(W):
        return W.T

    EH = E // 2
    gather = _make_gather(EH, 80)
    edge = _make_edge_call(EH, 1600)
    rbf_b = rbf.astype(bf)
    edge_w = (
        t(msW1)[:F].astype(bf), t(msW1)[F:].astype(bf), msb1.reshape(1, F),
        t(msW2).astype(bf), msb2.reshape(1, F),
        t(mvW1)[:F].astype(bf), t(mvW1)[F:].astype(bf), mvb1.reshape(1, F),
        t(mvW2).astype(bf), mvb2.reshape(1, F),
    )
    halves = []
    for h in range(2):
        sl = slice(h * EH, (h + 1) * EH)
        rowdat, coldat = gather(tbl, row[sl], col[sl])
        halves.append(edge(rowdat, coldat, rbf_b[sl], *edge_w))

    zero = jnp.zeros(((N // NS) // 8 * 8, F), jnp.float32)
    a0, a1, a2, a3 = _make_scatter(EH, N, 128)(
        halves[0][0], halves[1][0], halves[0][1], halves[1][1],
        halves[0][2], halves[1][2], halves[0][3], halves[1][3],
        col[:EH], col[EH:], zero)

    s_out, v2_out = _make_node_call(N, 2000)(
        s, v2, a0, a1, a2, a3,
        t(usW1)[:F], t(usW1)[F:], usb1.reshape(1, F), t(usW2), usb2.reshape(1, F),
        t(uvW1)[:F], t(uvW1)[F:], uvb1.reshape(1, F), t(uvW2), uvb2.reshape(1, F),
    )
    return s_out, v2_out.reshape(N, 3, F)
